# Initial kernel scaffold; baseline (speedup 1.0000x reference)
import jax
import jax.numpy as jnp
from jax.experimental import pallas as pl


def kernel(feat, coords, knn_idx, W, b):
    raise NotImplementedError("write your pallas kernel here")


# speedup vs baseline: 2.3333x; 2.3333x over previous
import functools

import jax
import jax.numpy as jnp
from jax import lax
from jax.experimental import pallas as pl
from jax.experimental.pallas import tpu as pltpu
from jax.experimental.pallas import tpu_sc as plsc

N = 10000
K = 32
C = 128
AW = 144
NC = 2
NS = 16
NW = NC * NS
NPAD = 10240
PW = NPAD // NW
BN = 4
KB = BN * K
NB = PW // BN


def _sc_body(aug_h, knn_h, outf_h, outs_h, idx_v, rows_v, accf_v, accs_v,
             sem0, sem1):
    cid = lax.axis_index("c")
    sid = lax.axis_index("s")
    wid = sid * NC + cid
    base = wid * PW

    pltpu.sync_copy(knn_h.at[pl.ds(wid * NB, NB)], idx_v)

    sems = (sem0, sem1)

    def issue(j, s):
        pltpu.async_copy(aug_h.at[idx_v.at[j]], rows_v.at[s], sems[s])

    issue(0, 0)

    def body(i, carry):
        j0 = 2 * i
        for bslot in (0, 1):
            j = j0 + bslot

            @pl.when(j + 1 < NB)
            def _():
                issue(j + 1, 1 - bslot)

            pltpu.make_async_copy(
                aug_h.at[idx_v.at[j]], rows_v.at[bslot], sems[bslot]).wait()

            node = j * BN
            for n in range(BN):
                r0 = n * K
                for c in range(AW // 16):
                    acc = rows_v[bslot, r0, pl.ds(c * 16, 16)]
                    for k in range(1, K):
                        acc = acc + rows_v[bslot, r0 + k, pl.ds(c * 16, 16)]
                    acc = acc * (1.0 / K)
                    if c < C // 16:
                        accf_v[node + n, pl.ds(c * 16, 16)] = acc
                    else:
                        accs_v[node + n, :] = acc
        return carry

    lax.fori_loop(0, NB // 2, body, 0)

    pltpu.sync_copy(accf_v, outf_h.at[pl.ds(base, PW)])
    pltpu.sync_copy(accs_v, outs_h.at[pl.ds(base, PW)])


def _sc_gather(aug, knn2d):
    mesh = plsc.VectorSubcoreMesh(
        core_axis_name="c", subcore_axis_name="s",
        num_cores=NC, num_subcores=NS)
    fn = pl.kernel(
        _sc_body,
        out_type=(
            jax.ShapeDtypeStruct((NPAD, C), jnp.float32),
            jax.ShapeDtypeStruct((NPAD, 16), jnp.float32),
        ),
        mesh=mesh,
        compiler_params=pltpu.CompilerParams(use_tc_tiling_on_sc=False),
        scratch_types=[
            pltpu.VMEM((NB, KB), jnp.int32),
            pltpu.VMEM((2, KB, AW), jnp.float32),
            pltpu.VMEM((PW, C), jnp.float32),
            pltpu.VMEM((PW, 16), jnp.float32),
            pltpu.SemaphoreType.DMA,
            pltpu.SemaphoreType.DMA,
        ],
    )
    return fn(aug, knn2d)


def _tc_body(feat_ref, sf_ref, ss_ref, c_ref, w1_ref, w2_ref, wa_ref, wb_ref,
             p_ref, b_ref, o_ref):
    t = ss_ref[...]
    ex = t
    ex2 = jnp.dot(t, p_ref[...], preferred_element_type=jnp.float32)
    mrel = ex - c_ref[...]
    var = jnp.maximum(ex2 - ex * ex, 0.0)
    std = jnp.sqrt(var)
    acc = jnp.dot(feat_ref[...], w1_ref[...], preferred_element_type=jnp.float32)
    acc = acc + jnp.dot(sf_ref[...], w2_ref[...], preferred_element_type=jnp.float32)
    acc = acc + jnp.dot(mrel, wa_ref[...], preferred_element_type=jnp.float32)
    acc = acc + jnp.dot(std, wb_ref[...], preferred_element_type=jnp.float32)
    acc = acc + b_ref[...]
    o_ref[...] = jnp.maximum(acc, 0.0)


def _tc_mix(feat, sfeat, sstat, cpad, w1, w2, wa, wb, p, b2):
    B = 1000
    grid = (N // B,)
    return pl.pallas_call(
        _tc_body,
        grid=grid,
        in_specs=[
            pl.BlockSpec((B, C), lambda i: (i, 0)),
            pl.BlockSpec((B, C), lambda i: (i, 0)),
            pl.BlockSpec((B, 16), lambda i: (i, 0)),
            pl.BlockSpec((B, 16), lambda i: (i, 0)),
            pl.BlockSpec((C, C), lambda i: (0, 0)),
            pl.BlockSpec((C, C), lambda i: (0, 0)),
            pl.BlockSpec((16, C), lambda i: (0, 0)),
            pl.BlockSpec((16, C), lambda i: (0, 0)),
            pl.BlockSpec((16, 16), lambda i: (0, 0)),
            pl.BlockSpec((1, C), lambda i: (0, 0)),
        ],
        out_specs=pl.BlockSpec((B, C), lambda i: (i, 0)),
        out_shape=jax.ShapeDtypeStruct((N, C), jnp.float32),
    )(feat, sfeat, sstat, cpad, w1, w2, wa, wb, p, b2)


def kernel(feat, coords, knn_idx, W, b):
    f32 = jnp.float32
    feat = feat.astype(f32)
    coords = coords.astype(f32)

    aug = jnp.concatenate(
        [feat, coords, coords * coords,
         jnp.zeros((N, AW - C - 6), f32)], axis=1)

    knn = knn_idx.astype(jnp.int32)
    knn = jnp.pad(knn, ((0, NPAD - N), (0, 0)))
    knn2d = knn.reshape(NPAD * K // KB, KB)

    sfeat, sstat = _sc_gather(aug, knn2d)

    w1 = W[0:C]
    w2 = W[C:2 * C]
    wa = jnp.zeros((16, C), f32).at[0:3].set(W[2 * C:2 * C + 3])
    wb = jnp.zeros((16, C), f32).at[0:3].set(W[2 * C + 3:2 * C + 6])
    p = jnp.zeros((16, 16), f32).at[jnp.arange(3) + 3, jnp.arange(3)].set(1.0)
    cpad = jnp.pad(coords, ((0, 0), (0, 13)))
    b2 = b.astype(f32).reshape(1, C)

    return _tc_mix(feat, sfeat[:N], sstat[:N], cpad, w1, w2, wa, wb, p, b2)
